# Initial kernel scaffold; baseline (speedup 1.0000x reference)
import jax
import jax.numpy as jnp
from jax.experimental import pallas as pl


def kernel(x, edge_index, W1, b1, W2, b2, Wl, bl):
    raise NotImplementedError("write your pallas kernel here")


# speedup vs baseline: 30.7594x; 30.7594x over previous
import functools

import jax
import jax.numpy as jnp
from jax import lax
from jax.experimental import pallas as pl
from jax.experimental.pallas import tpu as pltpu
from jax.experimental.pallas import tpu_sc as plsc

N = 10000
F = 128
E = 320000
NC = 2
NS = 16
NW = NC * NS
C = 128
NCH = 80
EPT = NCH * C
E_PAD = NW * EPT
NP = 10240
RPT = NP // NS


def _deg_body(dstp, degp, dacc, dst_v, ones_v, zeros_v):
    c = lax.axis_index("c")
    s = lax.axis_index("s")
    w = s * NC + c
    pltpu.sync_copy(dstp.at[w], dst_v)
    for i in range(C // 16):
        ones_v[pl.ds(i * 16, 16)] = jnp.ones((16,), jnp.float32)
    for i in range(RPT // 16):
        zeros_v[pl.ds(i * 16, 16)] = jnp.zeros((16,), jnp.float32)
    pltpu.sync_copy(zeros_v, dacc.at[pl.ds(s * RPT, RPT)])
    plsc.subcore_barrier()

    @pl.loop(0, NCH)
    def _(j):
        pltpu.sync_copy(ones_v, dacc.at[dst_v.at[j]], add=True)

    plsc.subcore_barrier()
    pltpu.sync_copy(dacc.at[pl.ds(s * RPT, RPT)],
                    degp.at[c, pl.ds(s * RPT, RPT)])


@functools.lru_cache(maxsize=None)
def _deg_kernel():
    mesh = plsc.VectorSubcoreMesh(core_axis_name="c", subcore_axis_name="s",
                                  num_cores=NC, num_subcores=NS)
    return pl.kernel(
        _deg_body,
        out_type=jax.ShapeDtypeStruct((NC, NP), jnp.float32),
        mesh=mesh,
        scratch_types=[
            pltpu.VMEM_SHARED((NP,), jnp.float32),
            pltpu.VMEM((NCH, C), jnp.int32),
            pltpu.VMEM((C,), jnp.float32),
            pltpu.VMEM((RPT,), jnp.float32),
        ],
    )


def _agg_body(xs, pack, z2d, part,
              acc, idx0, idx1, rows0, rows1, semi0, semi1, semg0, semg1):
    c = lax.axis_index("c")
    s = lax.axis_index("s")
    w = s * NC + c
    pltpu.sync_copy(z2d, acc.at[pl.ds(s * RPT, RPT)])
    plsc.subcore_barrier()

    pltpu.async_copy(pack.at[w, 0], idx0, semi0)
    pltpu.async_copy(pack.at[w, 1], idx1, semi1)
    pltpu.make_async_copy(pack.at[w, 0], idx0, semi0).wait()
    pltpu.async_copy(xs.at[idx0.at[0]], rows0, semg0)

    @pl.loop(0, NCH // 2)
    def _(j2):
        j = j2 * 2
        pltpu.make_async_copy(pack.at[w, 0], idx1, semi1).wait()
        pltpu.async_copy(xs.at[idx1.at[0]], rows1, semg1)

        pltpu.make_async_copy(xs.at[idx0.at[0]], rows0, semg0).wait()
        pltpu.sync_copy(rows0, acc.at[idx0.at[1]], add=True)

        @pl.when(j2 < NCH // 2 - 1)
        def _():
            pltpu.async_copy(pack.at[w, j + 2], idx0, semi0)
            pltpu.make_async_copy(pack.at[w, 0], idx0, semi0).wait()
            pltpu.async_copy(xs.at[idx0.at[0]], rows0, semg0)

        pltpu.make_async_copy(xs.at[idx1.at[0]], rows1, semg1).wait()
        pltpu.sync_copy(rows1, acc.at[idx1.at[1]], add=True)

        @pl.when(j2 < NCH // 2 - 1)
        def _():
            pltpu.async_copy(pack.at[w, j + 3], idx1, semi1)

    plsc.subcore_barrier()
    pltpu.sync_copy(acc.at[pl.ds(s * RPT, RPT)],
                    part.at[c, pl.ds(s * RPT, RPT)])


@functools.lru_cache(maxsize=None)
def _agg_kernel():
    mesh = plsc.VectorSubcoreMesh(core_axis_name="c", subcore_axis_name="s",
                                  num_cores=NC, num_subcores=NS)
    return pl.kernel(
        _agg_body,
        out_type=jax.ShapeDtypeStruct((NC, NP, F), jnp.float32),
        mesh=mesh,
        scratch_types=[
            pltpu.VMEM_SHARED((NP, F), jnp.float32),
            pltpu.VMEM((2, C), jnp.int32),
            pltpu.VMEM((2, C), jnp.int32),
            pltpu.VMEM((C, F), jnp.float32),
            pltpu.VMEM((C, F), jnp.float32),
            pltpu.SemaphoreType.DMA,
            pltpu.SemaphoreType.DMA,
            pltpu.SemaphoreType.DMA,
            pltpu.SemaphoreType.DMA,
        ],
    )


_BM = 512
_GRID = NP // _BM


def _scale_body(degp, x, dinv, xs):
    d = degp[0] + degp[1] + 1.0
    dv = lax.rsqrt(d)
    dinv[...] = dv
    xs[...] = x[...] * dv


def _scale_call(degp, x):
    return pl.pallas_call(
        _scale_body,
        grid=(_GRID,),
        in_specs=[
            pl.BlockSpec((NC, _BM, 1), lambda i: (0, i, 0)),
            pl.BlockSpec((_BM, F), lambda i: (i, 0)),
        ],
        out_specs=[
            pl.BlockSpec((_BM, 1), lambda i: (i, 0)),
            pl.BlockSpec((_BM, F), lambda i: (i, 0)),
        ],
        out_shape=[
            jax.ShapeDtypeStruct((NP, 1), jnp.float32),
            jax.ShapeDtypeStruct((NP, F), jnp.float32),
        ],
    )(degp, x)


def _mid_body(part, xs, dinv, W1, b1, W2, ts):
    agg = (part[0] + part[1] + xs[...]) * dinv[...]
    h1 = jnp.maximum(
        jnp.dot(agg, W1[...], preferred_element_type=jnp.float32) + b1[...],
        0.0)
    t = jnp.dot(h1, W2[...], preferred_element_type=jnp.float32)
    ts[...] = t * dinv[...]


def _mid_call(part, xs, dinv, W1, b1, W2):
    return pl.pallas_call(
        _mid_body,
        grid=(_GRID,),
        in_specs=[
            pl.BlockSpec((NC, _BM, F), lambda i: (0, i, 0)),
            pl.BlockSpec((_BM, F), lambda i: (i, 0)),
            pl.BlockSpec((_BM, 1), lambda i: (i, 0)),
            pl.BlockSpec((F, 2 * F), lambda i: (0, 0)),
            pl.BlockSpec((1, 2 * F), lambda i: (0, 0)),
            pl.BlockSpec((2 * F, F), lambda i: (0, 0)),
        ],
        out_specs=pl.BlockSpec((_BM, F), lambda i: (i, 0)),
        out_shape=jax.ShapeDtypeStruct((NP, F), jnp.float32),
    )(part, xs, dinv, W1, b1, W2)


def _fin_body(part, ts, dinv, b2, Wl, bl, out):
    h2 = jnp.maximum((part[0] + part[1] + ts[...]) * dinv[...] + b2[...], 0.0)
    lg = jnp.dot(h2, Wl[...], preferred_element_type=jnp.float32) + bl[...]
    m = jnp.max(lg, axis=-1, keepdims=True)
    lse = m + jnp.log(jnp.sum(jnp.exp(lg - m), axis=-1, keepdims=True))
    out[...] = lg - lse


def _fin_call(part, ts, dinv, b2, Wl, bl):
    return pl.pallas_call(
        _fin_body,
        grid=(_GRID,),
        in_specs=[
            pl.BlockSpec((NC, _BM, F), lambda i: (0, i, 0)),
            pl.BlockSpec((_BM, F), lambda i: (i, 0)),
            pl.BlockSpec((_BM, 1), lambda i: (i, 0)),
            pl.BlockSpec((1, F), lambda i: (0, 0)),
            pl.BlockSpec((F, 2), lambda i: (0, 0)),
            pl.BlockSpec((1, 2), lambda i: (0, 0)),
        ],
        out_specs=pl.BlockSpec((_BM, 2), lambda i: (i, 0)),
        out_shape=jax.ShapeDtypeStruct((NP, 2), jnp.float32),
    )(part, ts, dinv, b2, Wl, bl)


def kernel(x, edge_index, W1, b1, W2, b2, Wl, bl):
    src = edge_index[0]
    dst = edge_index[1]
    pad = E_PAD - E
    ar = jnp.arange(pad, dtype=jnp.int32)
    srcp = jnp.concatenate([src, ar % 128]).reshape(NW, NCH, 1, C)
    dstp = jnp.concatenate([dst, N + (ar % 128)]).reshape(NW, NCH, 1, C)
    pack = jnp.concatenate([srcp, dstp], axis=2)

    xp = jnp.zeros((NP, F), jnp.float32).at[:N].set(x)
    z2d = jnp.zeros((RPT, F), jnp.float32)

    degp = _deg_kernel()(dstp.reshape(NW, NCH, C))
    dinv, xs = _scale_call(degp.reshape(NC, NP, 1), xp)

    part1 = _agg_kernel()(xs, pack, z2d)
    ts = _mid_call(part1, xs, dinv, W1, b1.reshape(1, 2 * F), W2)

    part2 = _agg_kernel()(ts, pack, z2d)
    outp = _fin_call(part2, ts, dinv, b2.reshape(1, F), Wl, bl.reshape(1, 2))
    return outp[:N]
